# baseline (speedup 1.0000x reference)
import functools
from itertools import combinations

import numpy as np
import jax
import jax.numpy as jnp
from jax import lax
from jax.experimental import pallas as pl
from jax.experimental.pallas import tpu as pltpu
from jax.experimental.pallas import tpu_sc as plsc

_B, _F, _V, _D, _H = 4096, 26, 100000, 16, 128
_PAIRS = np.array(list(combinations(range(_F), 2)), dtype=np.int32)
_NPAIR = _PAIRS.shape[0]
_PI = _PAIRS[:, 0]
_PJ = _PAIRS[:, 1]

_NC, _NS, _L = 2, 16, 16
_NW = _NC * _NS
_ROWS = _B * _F
_NR = _ROWS // _NW
_CH = 128
_NCHUNK = _NR // _CH
_NVEC = _NR // _L


@functools.lru_cache(maxsize=None)
def _build_gather():
    mesh = plsc.VectorSubcoreMesh(core_axis_name="c", subcore_axis_name="s")

    @functools.partial(
        pl.kernel,
        mesh=mesh,
        out_type=jax.ShapeDtypeStruct((_ROWS, _D), jnp.float32),
        scratch_types=[
            pltpu.VMEM((_NR,), jnp.int32),
            pltpu.VMEM((_NR, _D), jnp.float32),
            pltpu.SemaphoreType.DMA,
        ],
        compiler_params=pltpu.CompilerParams(use_tc_tiling_on_sc=False),
    )
    def gather_k(xi_hbm, table_hbm, out_hbm, idx_v, rows_v, sem):
        wid = lax.axis_index("s") * _NC + lax.axis_index("c")
        base = wid * _NR
        pltpu.sync_copy(xi_hbm.at[pl.ds(base, _NR)], idx_v)

        def ibody(k, carry):
            sl = pl.ds(k * _L, _L)
            pos = k * _L + lax.iota(jnp.int32, _L)
            idx_v[sl] = idx_v[sl] + lax.rem(pos, _F) * _V
            return carry

        lax.fori_loop(0, _NVEC, ibody, 0)

        copies = []
        for c in range(_NCHUNK):
            copies.append(
                pltpu.async_copy(
                    table_hbm.at[idx_v.at[pl.ds(c * _CH, _CH)]],
                    rows_v.at[pl.ds(c * _CH, _CH)],
                    sem,
                )
            )
        for cp in copies:
            cp.wait()
        pltpu.sync_copy(rows_v, out_hbm.at[pl.ds(base, _NR)])

    return gather_k


def _dense_body(e_ref, xvr_ref, xv_ref, M_ref, W1_ref, b1_ref, w2h_ref,
                b2_ref, o_ref):
    es = e_ref[...] * xvr_ref[...]
    a = jnp.dot(es, M_ref[...], preferred_element_type=jnp.float32)
    q = jnp.sum(es * a, axis=1, keepdims=True)
    h = jnp.dot(xv_ref[...], W1_ref[...], preferred_element_type=jnp.float32)
    h = jnp.maximum(h + b1_ref[...], 0.0)
    o_ref[...] = q + jnp.sum(h * w2h_ref[...], axis=1, keepdims=True) \
        + b2_ref[...]


_BLK = 512
_FD = _F * _D


@functools.lru_cache(maxsize=None)
def _build_dense():
    return pl.pallas_call(
        _dense_body,
        grid=(_B // _BLK,),
        in_specs=[
            pl.BlockSpec((_BLK, _FD), lambda i: (i, 0)),
            pl.BlockSpec((_BLK, _FD), lambda i: (i, 0)),
            pl.BlockSpec((_BLK, _F), lambda i: (i, 0)),
            pl.BlockSpec((_FD, _FD), lambda i: (0, 0)),
            pl.BlockSpec((_F, _H), lambda i: (0, 0)),
            pl.BlockSpec((1, _H), lambda i: (0, 0)),
            pl.BlockSpec((1, _H), lambda i: (0, 0)),
            pl.BlockSpec((1, 1), lambda i: (0, 0)),
        ],
        out_specs=pl.BlockSpec((_BLK, 1), lambda i: (i, 0)),
        out_shape=jax.ShapeDtypeStruct((_B, 1), jnp.float32),
    )


def kernel(xv, xi, tables, W1, b1, W2, b2):
    xi32 = xi.astype(jnp.int32).reshape(_ROWS)
    tflat = tables.reshape(_F * _V, _D)
    e = _build_gather()(xi32, tflat)
    e2 = e.reshape(_B, _FD)
    xvr = jnp.repeat(xv, _D, axis=1)

    w2q = W2[:_NPAIR, 0] * 0.5
    m26 = (jnp.zeros((_F, _F), jnp.float32)
           .at[_PI, _PJ].set(w2q).at[_PJ, _PI].set(w2q))
    m = jnp.kron(m26, jnp.eye(_D, dtype=jnp.float32))

    return _build_dense()(
        e2, xvr, xv, m, W1,
        b1.reshape(1, _H),
        W2[_NPAIR:, 0].reshape(1, _H),
        b2.reshape(1, 1),
    )


# speedup vs baseline: 3.4231x; 3.4231x over previous
import functools
from itertools import combinations

import numpy as np
import jax
import jax.numpy as jnp
from jax import lax
from jax.experimental import pallas as pl
from jax.experimental.pallas import tpu as pltpu
from jax.experimental.pallas import tpu_sc as plsc

_B, _F, _V, _D, _H = 4096, 26, 100000, 16, 128
_PAIRS = np.array(list(combinations(range(_F), 2)), dtype=np.int32)
_NPAIR = _PAIRS.shape[0]
_PI = _PAIRS[:, 0]
_PJ = _PAIRS[:, 1]

_NC, _NS, _L = 2, 16, 16
_NW = _NC * _NS
_ROWS = _B * _F
_NR = _ROWS // _NW
_NVEC = _NR // _L
_QW = _NR // 8
_LAG = 16


@functools.lru_cache(maxsize=None)
def _build_gather():
    mesh = plsc.VectorSubcoreMesh(core_axis_name="c", subcore_axis_name="s")

    @functools.partial(
        pl.kernel,
        mesh=mesh,
        out_type=jax.ShapeDtypeStruct((_ROWS // 8, 128), jnp.float32),
        scratch_types=[
            pltpu.VMEM((_NR,), jnp.int32),
            pltpu.VMEM((_QW, 128), jnp.float32),
            pltpu.SemaphoreType.DMA,
        ],
    )
    def gather_k(xi_hbm, t2d_hbm, out_hbm, idx_v, pack_v, sem):
        wid = lax.axis_index("s") * _NC + lax.axis_index("c")
        base = wid * _NR
        pltpu.sync_copy(xi_hbm.at[pl.ds(base, _NR)], idx_v)

        def ibody(k, carry):
            sl = pl.ds(k * _L, _L)
            pos = k * _L + lax.iota(jnp.int32, _L)
            idx_v[sl] = idx_v[sl] + lax.rem(pos, _F) * _V
            return carry

        lax.fori_loop(0, _NVEC, ibody, 0)

        def drain(q0):
            pltpu.make_async_copy(
                out_hbm.at[pl.ds(0, 2)], pack_v.at[pl.ds(q0, 2)], sem
            ).wait()

        def gbody(g, carry):
            ivec = idx_v[pl.ds(g * _L, _L)]
            for j in range(_L):
                r = ivec[j]
                q = g * 2 + j // 8
                s16 = (j % 8) * _D
                pltpu.async_copy(
                    t2d_hbm.at[r], pack_v.at[q, pl.ds(s16, _D)], sem)

            @pl.when(g >= _LAG)
            def _():
                drain((g - _LAG) * 2)

            return carry

        lax.fori_loop(0, _NVEC, gbody, 0)
        for w in range(_LAG):
            drain((_NVEC - _LAG + w) * 2)
        pltpu.sync_copy(pack_v, out_hbm.at[pl.ds(wid * _QW, _QW)])

    return gather_k


def _dense_body(e_ref, xvr_ref, xv_ref, M_ref, W1_ref, b1_ref, w2h_ref,
                b2_ref, o_ref):
    es = e_ref[...] * xvr_ref[...]
    a = jnp.dot(es, M_ref[...], preferred_element_type=jnp.float32)
    q = jnp.sum(es * a, axis=1, keepdims=True)
    h = jnp.dot(xv_ref[...], W1_ref[...], preferred_element_type=jnp.float32)
    h = jnp.maximum(h + b1_ref[...], 0.0)
    o_ref[...] = q + jnp.sum(h * w2h_ref[...], axis=1, keepdims=True) \
        + b2_ref[...]


_BLK = 512
_FD = _F * _D


@functools.lru_cache(maxsize=None)
def _build_dense():
    return pl.pallas_call(
        _dense_body,
        grid=(_B // _BLK,),
        in_specs=[
            pl.BlockSpec((_BLK, _FD), lambda i: (i, 0)),
            pl.BlockSpec((_BLK, _FD), lambda i: (i, 0)),
            pl.BlockSpec((_BLK, _F), lambda i: (i, 0)),
            pl.BlockSpec((_FD, _FD), lambda i: (0, 0)),
            pl.BlockSpec((_F, _H), lambda i: (0, 0)),
            pl.BlockSpec((1, _H), lambda i: (0, 0)),
            pl.BlockSpec((1, _H), lambda i: (0, 0)),
            pl.BlockSpec((1, 1), lambda i: (0, 0)),
        ],
        out_specs=pl.BlockSpec((_BLK, 1), lambda i: (i, 0)),
        out_shape=jax.ShapeDtypeStruct((_B, 1), jnp.float32),
    )


def kernel(xv, xi, tables, W1, b1, W2, b2):
    xi32 = xi.astype(jnp.int32).reshape(_ROWS)
    t2d = tables.reshape(_F * _V, _D)
    e = _build_gather()(xi32, t2d)
    e2 = e.reshape(_B, _FD)
    xvr = jnp.repeat(xv, _D, axis=1)

    w2q = W2[:_NPAIR, 0] * 0.5
    m26 = (jnp.zeros((_F, _F), jnp.float32)
           .at[_PI, _PJ].set(w2q).at[_PJ, _PI].set(w2q))
    m = jnp.kron(m26, jnp.eye(_D, dtype=jnp.float32))

    return _build_dense()(
        e2, xvr, xv, m, W1,
        b1.reshape(1, _H),
        W2[_NPAIR:, 0].reshape(1, _H),
        b2.reshape(1, 1),
    )
